# baseline (speedup 1.0000x reference)
import functools

import jax
import jax.numpy as jnp
from jax import lax
from jax.experimental import pallas as pl
from jax.experimental.pallas import tpu as pltpu
from jax.experimental.pallas import tpu_sc as plsc

BATCH = 64
SEQ = 50
RANK = 64
VOCAB = 1000

NC = 2
NS = 16
NW = NC * NS
ROWS_PER_W = BATCH // NW
CHUNK = 8
NCHUNK = (SEQ + CHUNK - 1) // CHUNK
SEQ_PAD = NCHUNK * CHUNK
NBLK = RANK // 16


_VB = 40


def _sq_transpose_body(c_ref, t_ref):
    for k in range(_VB):
        blk = c_ref[:, k, :]
        t_ref[k] = blk * blk


def _build_table(core3):
    return pl.pallas_call(
        _sq_transpose_body,
        grid=(VOCAB // _VB,),
        in_specs=[pl.BlockSpec((RANK, _VB, RANK), lambda i: (0, i, 0))],
        out_specs=pl.BlockSpec((_VB, RANK, RANK), lambda i: (i, 0, 0)),
        out_shape=jax.ShapeDtypeStruct((VOCAB, RANK, RANK), jnp.float32),
    )(core3)


def _sc_body(table, y, alpha, beta, out, y_v, idx_v, a_v, b_v, buf, out_v,
             sem):
    wid = lax.axis_index("s") * NC + lax.axis_index("c")
    pltpu.sync_copy(alpha, a_v)
    pltpu.sync_copy(beta, b_v)
    lanes = lax.broadcasted_iota(jnp.int32, (16,), 0)
    out_vec = jnp.zeros((16,), jnp.float32)

    for row_i in range(ROWS_PER_W):
        b = wid * ROWS_PER_W + row_i
        pltpu.sync_copy(y.at[pl.ds(b * SEQ_PAD, SEQ_PAD)],
                        y_v.at[pl.ds(0, SEQ_PAD)])

        cr0 = []
        for blk in range(NBLK):
            a = a_v[pl.ds(blk * 16, 16)]
            cr0.append(a * a)

        def chunk_body(c, cr):
            idx_v[...] = y_v[pl.ds(c * CHUNK, 16)]
            pltpu.async_copy(table.at[idx_v.at[pl.ds(0, CHUNK)]], buf,
                             sem).wait()
            nsteps = jnp.where(c == NCHUNK - 1, SEQ - (NCHUNK - 1) * CHUNK,
                               CHUNK)

            def step(j, cr):
                acc = [jnp.zeros((16,), jnp.float32) for _ in range(NBLK)]
                for rb in range(NBLK):
                    cvec = cr[rb]
                    for rl in range(16):
                        r = rb * 16 + rl
                        cs = jnp.sum(jnp.where(lanes == rl, cvec, 0.0))
                        cb = jnp.full((16,), cs)
                        for blk in range(NBLK):
                            m = buf[j, pl.ds(r * RANK + blk * 16, 16)]
                            acc[blk] = acc[blk] + cb * m
                return tuple(acc)

            return lax.fori_loop(0, nsteps, step, cr)

        cr = lax.fori_loop(0, NCHUNK, chunk_body, tuple(cr0))

        tot = jnp.zeros((16,), jnp.float32)
        for blk in range(NBLK):
            bb = b_v[pl.ds(blk * 16, 16)]
            tot = tot + cr[blk] * bb * bb
        total = jnp.sum(tot)
        out_vec = jnp.where(lanes == row_i, jnp.full((16,), total), out_vec)

    out_v[...] = out_vec
    pltpu.sync_copy(out_v, out.at[pl.ds(wid * 16, 16)])


@functools.partial(
    pl.kernel,
    out_type=jax.ShapeDtypeStruct((NW * 16,), jnp.float32),
    mesh=plsc.VectorSubcoreMesh(
        core_axis_name="c", subcore_axis_name="s", num_cores=NC,
        num_subcores=NS),
    compiler_params=pltpu.CompilerParams(needs_layout_passes=False,
                                         use_tc_tiling_on_sc=True),
    scratch_types=[
        pltpu.VMEM((64,), jnp.int32),
        pltpu.VMEM((16,), jnp.int32),
        pltpu.VMEM((RANK,), jnp.float32),
        pltpu.VMEM((RANK,), jnp.float32),
        pltpu.VMEM((CHUNK, RANK * RANK), jnp.float32),
        pltpu.VMEM((16,), jnp.float32),
        pltpu.SemaphoreType.DMA,
    ],
)
def _sc_chain(table, y, alpha, beta, out, *scratch):
    _sc_body(table, y, alpha, beta, out, *scratch)


def kernel(y, alpha, beta, core):
    table = _build_table(core[0]).reshape(VOCAB, RANK * RANK)
    y_pad = jnp.pad(y.astype(jnp.int32), ((0, 0), (0, SEQ_PAD - SEQ)))
    out_flat = _sc_chain(table, y_pad.reshape(-1), alpha[0], beta[0])
    return out_flat.reshape(NW, 16)[:, :ROWS_PER_W].reshape(BATCH)


# speedup vs baseline: 1.3650x; 1.3650x over previous
import functools

import jax
import jax.numpy as jnp
from jax import lax
from jax.experimental import pallas as pl
from jax.experimental.pallas import tpu as pltpu
from jax.experimental.pallas import tpu_sc as plsc

BATCH = 64
SEQ = 50
RANK = 64
VOCAB = 1000

NC = 2
NS = 16
NW = NC * NS
ROWS_PER_W = BATCH // NW
CHUNK = 8
NCHUNK = (SEQ + CHUNK - 1) // CHUNK
SEQ_PAD = NCHUNK * CHUNK
NBLK = RANK // 16


def _sq_transpose_body(c_ref, t_ref):
    a = c_ref[0]
    b = c_ref[1]
    t_ref[:, 0:RANK] = a * a
    t_ref[:, RANK:2 * RANK] = b * b


def _build_table(core3):
    return pl.pallas_call(
        _sq_transpose_body,
        grid=(RANK // 2,),
        in_specs=[pl.BlockSpec((2, VOCAB, RANK), lambda i: (i, 0, 0))],
        out_specs=pl.BlockSpec((VOCAB, 2 * RANK), lambda i: (0, i)),
        out_shape=jax.ShapeDtypeStruct((VOCAB, RANK * RANK), jnp.float32),
    )(core3)


def _sc_body(table, y, alpha, beta, out, y_v, idx_v, a_v, b_v, buf, out_v,
             sem):
    wid = lax.axis_index("s") * NC + lax.axis_index("c")
    pltpu.sync_copy(alpha, a_v)
    pltpu.sync_copy(beta, b_v)
    lanes = lax.broadcasted_iota(jnp.int32, (16,), 0)
    out_vec = jnp.zeros((16,), jnp.float32)

    for row_i in range(ROWS_PER_W):
        b = wid * ROWS_PER_W + row_i
        pltpu.sync_copy(y.at[pl.ds(b * SEQ_PAD, SEQ_PAD)],
                        y_v.at[pl.ds(0, SEQ_PAD)])

        cr0 = []
        for blk in range(NBLK):
            a = a_v[pl.ds(blk * 16, 16)]
            cr0.append(a * a)

        def chunk_body(c, cr):
            idx_v[...] = y_v[pl.ds(c * CHUNK, 16)]
            pltpu.async_copy(table.at[idx_v.at[pl.ds(0, CHUNK)]], buf,
                             sem).wait()
            nsteps = jnp.where(c == NCHUNK - 1, SEQ - (NCHUNK - 1) * CHUNK,
                               CHUNK)

            def step(j, cr):
                acc = [jnp.zeros((16,), jnp.float32) for _ in range(NBLK)]
                for rb in range(NBLK):
                    cvec = cr[rb]
                    for rl in range(16):
                        r = rb * 16 + rl
                        cs = jnp.sum(jnp.where(lanes == rl, cvec, 0.0))
                        cb = jnp.full((16,), cs)
                        for blk in range(NBLK):
                            m = buf[j, pl.ds(r * RANK + blk * 16, 16)]
                            acc[blk] = acc[blk] + cb * m
                return tuple(acc)

            return lax.fori_loop(0, nsteps, step, cr)

        cr = lax.fori_loop(0, NCHUNK, chunk_body, tuple(cr0))

        tot = jnp.zeros((16,), jnp.float32)
        for blk in range(NBLK):
            bb = b_v[pl.ds(blk * 16, 16)]
            tot = tot + cr[blk] * bb * bb
        total = jnp.sum(tot)
        out_vec = jnp.where(lanes == row_i, jnp.full((16,), total), out_vec)

    out_v[...] = out_vec
    pltpu.sync_copy(out_v, out.at[pl.ds(wid * 16, 16)])


@functools.partial(
    pl.kernel,
    out_type=jax.ShapeDtypeStruct((NW * 16,), jnp.float32),
    mesh=plsc.VectorSubcoreMesh(
        core_axis_name="c", subcore_axis_name="s", num_cores=NC,
        num_subcores=NS),
    compiler_params=pltpu.CompilerParams(needs_layout_passes=False,
                                         use_tc_tiling_on_sc=True),
    scratch_types=[
        pltpu.VMEM((64,), jnp.int32),
        pltpu.VMEM((16,), jnp.int32),
        pltpu.VMEM((RANK,), jnp.float32),
        pltpu.VMEM((RANK,), jnp.float32),
        pltpu.VMEM((CHUNK, RANK * RANK), jnp.float32),
        pltpu.VMEM((16,), jnp.float32),
        pltpu.SemaphoreType.DMA,
    ],
)
def _sc_chain(table, y, alpha, beta, out, *scratch):
    _sc_body(table, y, alpha, beta, out, *scratch)


def kernel(y, alpha, beta, core):
    table = _build_table(core[0])
    y_pad = jnp.pad(y.astype(jnp.int32), ((0, 0), (0, SEQ_PAD - SEQ)))
    out_flat = _sc_chain(table, y_pad.reshape(-1), alpha[0], beta[0])
    return out_flat.reshape(NW, 16)[:, :ROWS_PER_W].reshape(BATCH)


# speedup vs baseline: 1.4955x; 1.0956x over previous
import functools

import jax
import jax.numpy as jnp
from jax import lax
from jax.experimental import pallas as pl
from jax.experimental.pallas import tpu as pltpu
from jax.experimental.pallas import tpu_sc as plsc

BATCH = 64
SEQ = 50
RANK = 64
VOCAB = 1000

NC = 2
NS = 16
NW = NC * NS
ROWS_PER_W = BATCH // NW
CHUNK = 8
NCHUNK = (SEQ + CHUNK - 1) // CHUNK
SEQ_PAD = NCHUNK * CHUNK
NBLK = RANK // 16


def _sq_transpose_body(c_ref, t_ref):
    a = c_ref[0]
    b = c_ref[1]
    t_ref[:, 0:RANK] = a * a
    t_ref[:, RANK:2 * RANK] = b * b


def _build_table(core3):
    return pl.pallas_call(
        _sq_transpose_body,
        grid=(RANK // 2,),
        in_specs=[pl.BlockSpec((2, VOCAB, RANK), lambda i: (i, 0, 0))],
        out_specs=pl.BlockSpec((VOCAB, 2 * RANK), lambda i: (0, i)),
        out_shape=jax.ShapeDtypeStruct((VOCAB, RANK * RANK), jnp.float32),
    )(core3)


def _sc_body(table, y, alpha, beta, out, y_v, idx_v0, idx_v1, a_v, b_v, buf0,
             buf1, out_v, sem0, sem1):
    wid = lax.axis_index("s") * NC + lax.axis_index("c")
    pltpu.sync_copy(alpha, a_v)
    pltpu.sync_copy(beta, b_v)
    lanes = lax.broadcasted_iota(jnp.int32, (16,), 0)

    bufs = (buf0, buf1)
    idxs = (idx_v0, idx_v1)
    sems = (sem0, sem1)

    def step_on(buf, j, cr):
        acc = [[jnp.zeros((16,), jnp.float32) for _ in range(NBLK)]
               for _ in range(NBLK)]
        for rb in range(NBLK):
            cvec = cr[rb]
            for rl in range(16):
                r = rb * 16 + rl
                cs = jnp.sum(jnp.where(lanes == rl, cvec, 0.0))
                cb = jnp.full((16,), cs)
                for blk in range(NBLK):
                    m = buf[j, pl.ds(r * RANK + blk * 16, 16)]
                    acc[rb][blk] = acc[rb][blk] + cb * m
        return tuple((acc[0][blk] + acc[1][blk]) + (acc[2][blk] + acc[3][blk])
                     for blk in range(NBLK))

    def row_body(row_i, out_vec):
        b = wid * ROWS_PER_W + row_i
        pltpu.sync_copy(y.at[pl.ds(b * SEQ_PAD, SEQ_PAD)],
                        y_v.at[pl.ds(0, SEQ_PAD)])

        cr = []
        for blk in range(NBLK):
            a = a_v[pl.ds(blk * 16, 16)]
            cr.append(a * a)
        cr = tuple(cr)

        idx_v0[...] = y_v[pl.ds(0, 16)]
        pltpu.async_copy(table.at[idx_v0.at[pl.ds(0, CHUNK)]], buf0, sem0)
        for c in range(NCHUNK):
            if c + 1 < NCHUNK:
                nidx = idxs[(c + 1) % 2]
                nidx[...] = y_v[pl.ds((c + 1) * CHUNK, 16)]
                pltpu.async_copy(table.at[nidx.at[pl.ds(0, CHUNK)]],
                                 bufs[(c + 1) % 2], sems[(c + 1) % 2])
            pltpu.make_async_copy(table.at[idxs[c % 2].at[pl.ds(0, CHUNK)]],
                                  bufs[c % 2], sems[c % 2]).wait()
            nsteps = CHUNK if c + 1 < NCHUNK else SEQ - (NCHUNK - 1) * CHUNK
            cr = lax.fori_loop(
                0, nsteps,
                functools.partial(step_on, bufs[c % 2]), cr)

        tot = jnp.zeros((16,), jnp.float32)
        for blk in range(NBLK):
            bb = b_v[pl.ds(blk * 16, 16)]
            tot = tot + cr[blk] * bb * bb
        total = jnp.sum(tot)
        return jnp.where(lanes == row_i, jnp.full((16,), total), out_vec)

    out_vec = lax.fori_loop(0, ROWS_PER_W, row_body,
                            jnp.zeros((16,), jnp.float32))
    out_v[...] = out_vec
    pltpu.sync_copy(out_v, out.at[pl.ds(wid * 16, 16)])


@functools.partial(
    pl.kernel,
    out_type=jax.ShapeDtypeStruct((NW * 16,), jnp.float32),
    mesh=plsc.VectorSubcoreMesh(
        core_axis_name="c", subcore_axis_name="s", num_cores=NC,
        num_subcores=NS),
    compiler_params=pltpu.CompilerParams(needs_layout_passes=False,
                                         use_tc_tiling_on_sc=True),
    scratch_types=[
        pltpu.VMEM((64,), jnp.int32),
        pltpu.VMEM((16,), jnp.int32),
        pltpu.VMEM((16,), jnp.int32),
        pltpu.VMEM((RANK,), jnp.float32),
        pltpu.VMEM((RANK,), jnp.float32),
        pltpu.VMEM((CHUNK, RANK * RANK), jnp.float32),
        pltpu.VMEM((CHUNK, RANK * RANK), jnp.float32),
        pltpu.VMEM((16,), jnp.float32),
        pltpu.SemaphoreType.DMA,
        pltpu.SemaphoreType.DMA,
    ],
)
def _sc_chain(table, y, alpha, beta, out, *scratch):
    _sc_body(table, y, alpha, beta, out, *scratch)


def kernel(y, alpha, beta, core):
    table = _build_table(core[0])
    y_pad = jnp.pad(y.astype(jnp.int32), ((0, 0), (0, SEQ_PAD - SEQ)))
    out_flat = _sc_chain(table, y_pad.reshape(-1), alpha[0], beta[0])
    return out_flat.reshape(NW, 16)[:, :ROWS_PER_W].reshape(BATCH)


# speedup vs baseline: 1.6731x; 1.1188x over previous
import functools

import jax
import jax.numpy as jnp
from jax import lax
from jax.experimental import pallas as pl
from jax.experimental.pallas import tpu as pltpu
from jax.experimental.pallas import tpu_sc as plsc

BATCH = 64
SEQ = 50
RANK = 64
VOCAB = 1000

NC = 2
NS = 16
NW = NC * NS
ROWS_PER_W = BATCH // NW
CHUNK = 8
NCHUNK = (SEQ + CHUNK - 1) // CHUNK
SEQ_PAD = NCHUNK * CHUNK
NBLK = RANK // 16


def _sq_transpose_body(c_ref, t_ref):
    for q in range(2):
        x = c_ref[q]
        xt = jnp.transpose(x)
        t_ref[:, q * RANK:(q + 1) * RANK] = xt * xt


def _build_table(core_t):
    return pl.pallas_call(
        _sq_transpose_body,
        grid=(RANK // 2,),
        in_specs=[pl.BlockSpec((2, RANK, VOCAB), lambda i: (i, 0, 0))],
        out_specs=pl.BlockSpec((VOCAB, 2 * RANK), lambda i: (0, i)),
        out_shape=jax.ShapeDtypeStruct((VOCAB, RANK * RANK), jnp.float32),
    )(core_t)


def _sc_body(table, y, alpha, beta, out, y_v, idx_v0, idx_v1, a_v, b_v, buf0,
             buf1, out_v, sem0, sem1):
    wid = lax.axis_index("s") * NC + lax.axis_index("c")
    pltpu.sync_copy(alpha, a_v)
    pltpu.sync_copy(beta, b_v)
    lanes = lax.broadcasted_iota(jnp.int32, (16,), 0)

    bufs = (buf0, buf1)
    idxs = (idx_v0, idx_v1)
    sems = (sem0, sem1)

    def step_on(buf, j, cr):
        acc = [[jnp.zeros((16,), jnp.float32) for _ in range(NBLK)]
               for _ in range(NBLK)]
        for rb in range(NBLK):
            cvec = cr[rb]
            for rl in range(16):
                r = rb * 16 + rl
                cs = jnp.sum(jnp.where(lanes == rl, cvec, 0.0))
                cb = jnp.full((16,), cs)
                for blk in range(NBLK):
                    m = buf[j, pl.ds(r * RANK + blk * 16, 16)]
                    acc[rb][blk] = acc[rb][blk] + cb * m
        return tuple((acc[0][blk] + acc[1][blk]) + (acc[2][blk] + acc[3][blk])
                     for blk in range(NBLK))

    def row_body(row_i, out_vec):
        b = wid * ROWS_PER_W + row_i
        pltpu.sync_copy(y.at[pl.ds(b * SEQ_PAD, SEQ_PAD)],
                        y_v.at[pl.ds(0, SEQ_PAD)])

        cr = []
        for blk in range(NBLK):
            a = a_v[pl.ds(blk * 16, 16)]
            cr.append(a * a)
        cr = tuple(cr)

        idx_v0[...] = y_v[pl.ds(0, 16)]
        pltpu.async_copy(table.at[idx_v0.at[pl.ds(0, CHUNK)]], buf0, sem0)
        for c in range(NCHUNK):
            if c + 1 < NCHUNK:
                nidx = idxs[(c + 1) % 2]
                nidx[...] = y_v[pl.ds((c + 1) * CHUNK, 16)]
                pltpu.async_copy(table.at[nidx.at[pl.ds(0, CHUNK)]],
                                 bufs[(c + 1) % 2], sems[(c + 1) % 2])
            pltpu.make_async_copy(table.at[idxs[c % 2].at[pl.ds(0, CHUNK)]],
                                  bufs[c % 2], sems[c % 2]).wait()
            nsteps = CHUNK if c + 1 < NCHUNK else SEQ - (NCHUNK - 1) * CHUNK
            cr = lax.fori_loop(
                0, nsteps,
                functools.partial(step_on, bufs[c % 2]), cr)

        tot = jnp.zeros((16,), jnp.float32)
        for blk in range(NBLK):
            bb = b_v[pl.ds(blk * 16, 16)]
            tot = tot + cr[blk] * bb * bb
        total = jnp.sum(tot)
        return jnp.where(lanes == row_i, jnp.full((16,), total), out_vec)

    out_vec = lax.fori_loop(0, ROWS_PER_W, row_body,
                            jnp.zeros((16,), jnp.float32))
    out_v[...] = out_vec
    pltpu.sync_copy(out_v, out.at[pl.ds(wid * 16, 16)])


@functools.partial(
    pl.kernel,
    out_type=jax.ShapeDtypeStruct((NW * 16,), jnp.float32),
    mesh=plsc.VectorSubcoreMesh(
        core_axis_name="c", subcore_axis_name="s", num_cores=NC,
        num_subcores=NS),
    compiler_params=pltpu.CompilerParams(needs_layout_passes=False,
                                         use_tc_tiling_on_sc=True),
    scratch_types=[
        pltpu.VMEM((64,), jnp.int32),
        pltpu.VMEM((16,), jnp.int32),
        pltpu.VMEM((16,), jnp.int32),
        pltpu.VMEM((RANK,), jnp.float32),
        pltpu.VMEM((RANK,), jnp.float32),
        pltpu.VMEM((CHUNK, RANK * RANK), jnp.float32),
        pltpu.VMEM((CHUNK, RANK * RANK), jnp.float32),
        pltpu.VMEM((16,), jnp.float32),
        pltpu.SemaphoreType.DMA,
        pltpu.SemaphoreType.DMA,
    ],
)
def _sc_chain(table, y, alpha, beta, out, *scratch):
    _sc_body(table, y, alpha, beta, out, *scratch)


def kernel(y, alpha, beta, core):
    table = _build_table(jnp.swapaxes(core[0], 1, 2))
    y_pad = jnp.pad(y.astype(jnp.int32), ((0, 0), (0, SEQ_PAD - SEQ)))
    out_flat = _sc_chain(table, y_pad.reshape(-1), alpha[0], beta[0])
    return out_flat.reshape(NW, 16)[:, :ROWS_PER_W].reshape(BATCH)


# speedup vs baseline: 1.6810x; 1.0047x over previous
import functools

import jax
import jax.numpy as jnp
from jax import lax
from jax.experimental import pallas as pl
from jax.experimental.pallas import tpu as pltpu
from jax.experimental.pallas import tpu_sc as plsc

BATCH = 64
SEQ = 50
RANK = 64
VOCAB = 1000

NC = 2
NS = 16
NW = NC * NS
ROWS_PER_W = BATCH // NW
CHUNK = 8
NCHUNK = (SEQ + CHUNK - 1) // CHUNK
SEQ_PAD = NCHUNK * CHUNK
NBLK = RANK // 16


def _sq_transpose_body(c_ref, t_ref):
    for q in range(2):
        x = c_ref[q]
        xt = jnp.transpose(x)
        t_ref[:, q * RANK:(q + 1) * RANK] = xt * xt


def _build_table(core_t):
    return pl.pallas_call(
        _sq_transpose_body,
        grid=(RANK // 2,),
        in_specs=[pl.BlockSpec((2, RANK, VOCAB), lambda i: (i, 0, 0))],
        out_specs=pl.BlockSpec((VOCAB, 2 * RANK), lambda i: (0, i)),
        out_shape=jax.ShapeDtypeStruct((VOCAB, RANK * RANK), jnp.float32),
    )(core_t)


def _sc_body(table, y, alpha, beta, out, y_v, idx_v0, idx_v1, a_v, b_v, buf0,
             buf1, out_v, sem0, sem1):
    wid = lax.axis_index("s") * NC + lax.axis_index("c")
    pltpu.sync_copy(alpha, a_v)
    pltpu.sync_copy(beta, b_v)
    lanes = lax.broadcasted_iota(jnp.int32, (16,), 0)

    bufs = (buf0, buf1)
    idxs = (idx_v0, idx_v1)
    sems = (sem0, sem1)

    def step_on(buf, j, cr):
        acc = [[jnp.zeros((16,), jnp.float32) for _ in range(NBLK)]
               for _ in range(NBLK)]
        for rb in range(NBLK):
            cvec = cr[rb]
            cbs = []
            for rl in range(16):
                cs = jnp.sum(jnp.where(lanes == rl, cvec, 0.0))
                cbs.append(jnp.full((16,), cs))
            for rl in range(16):
                r = rb * 16 + rl
                for blk in range(NBLK):
                    m = buf[j, pl.ds(r * RANK + blk * 16, 16)]
                    acc[rb][blk] = acc[rb][blk] + cbs[rl] * m
        return tuple((acc[0][blk] + acc[1][blk]) + (acc[2][blk] + acc[3][blk])
                     for blk in range(NBLK))

    def row_body(row_i, out_vec):
        b = wid * ROWS_PER_W + row_i
        pltpu.sync_copy(y.at[pl.ds(b * SEQ_PAD, SEQ_PAD)],
                        y_v.at[pl.ds(0, SEQ_PAD)])

        cr = []
        for blk in range(NBLK):
            a = a_v[pl.ds(blk * 16, 16)]
            cr.append(a * a)
        cr = tuple(cr)

        idx_v0[...] = y_v[pl.ds(0, 16)]
        pltpu.async_copy(table.at[idx_v0.at[pl.ds(0, CHUNK)]], buf0, sem0)
        for c in range(NCHUNK):
            if c + 1 < NCHUNK:
                nidx = idxs[(c + 1) % 2]
                nidx[...] = y_v[pl.ds((c + 1) * CHUNK, 16)]
                pltpu.async_copy(table.at[nidx.at[pl.ds(0, CHUNK)]],
                                 bufs[(c + 1) % 2], sems[(c + 1) % 2])
            pltpu.make_async_copy(table.at[idxs[c % 2].at[pl.ds(0, CHUNK)]],
                                  bufs[c % 2], sems[c % 2]).wait()
            nsteps = CHUNK if c + 1 < NCHUNK else SEQ - (NCHUNK - 1) * CHUNK
            cr = lax.fori_loop(
                0, nsteps,
                functools.partial(step_on, bufs[c % 2]), cr)

        tot = jnp.zeros((16,), jnp.float32)
        for blk in range(NBLK):
            bb = b_v[pl.ds(blk * 16, 16)]
            tot = tot + cr[blk] * bb * bb
        total = jnp.sum(tot)
        return jnp.where(lanes == row_i, jnp.full((16,), total), out_vec)

    out_vec = lax.fori_loop(0, ROWS_PER_W, row_body,
                            jnp.zeros((16,), jnp.float32))
    out_v[...] = out_vec
    pltpu.sync_copy(out_v, out.at[pl.ds(wid * 16, 16)])


@functools.partial(
    pl.kernel,
    out_type=jax.ShapeDtypeStruct((NW * 16,), jnp.float32),
    mesh=plsc.VectorSubcoreMesh(
        core_axis_name="c", subcore_axis_name="s", num_cores=NC,
        num_subcores=NS),
    compiler_params=pltpu.CompilerParams(needs_layout_passes=False,
                                         use_tc_tiling_on_sc=True),
    scratch_types=[
        pltpu.VMEM((64,), jnp.int32),
        pltpu.VMEM((16,), jnp.int32),
        pltpu.VMEM((16,), jnp.int32),
        pltpu.VMEM((RANK,), jnp.float32),
        pltpu.VMEM((RANK,), jnp.float32),
        pltpu.VMEM((CHUNK, RANK * RANK), jnp.float32),
        pltpu.VMEM((CHUNK, RANK * RANK), jnp.float32),
        pltpu.VMEM((16,), jnp.float32),
        pltpu.SemaphoreType.DMA,
        pltpu.SemaphoreType.DMA,
    ],
)
def _sc_chain(table, y, alpha, beta, out, *scratch):
    _sc_body(table, y, alpha, beta, out, *scratch)


def kernel(y, alpha, beta, core):
    table = _build_table(jnp.swapaxes(core[0], 1, 2))
    y_pad = jnp.pad(y.astype(jnp.int32), ((0, 0), (0, SEQ_PAD - SEQ)))
    out_flat = _sc_chain(table, y_pad.reshape(-1), alpha[0], beta[0])
    return out_flat.reshape(NW, 16)[:, :ROWS_PER_W].reshape(BATCH)
